# Initial kernel scaffold; baseline (speedup 1.0000x reference)
import jax
import jax.numpy as jnp
from jax.experimental import pallas as pl


def kernel(x, batch):
    raise NotImplementedError("write your pallas kernel here")


# speedup vs baseline: 2.5827x; 2.5827x over previous
import jax
import jax.numpy as jnp
from jax import lax
from jax.experimental import pallas as pl
from jax.experimental.pallas import tpu as pltpu
from jax.experimental.pallas import tpu_sc as plsc

NUM_NODES = 100000
HIDDEN_DIM = 128
NUM_GRAPHS = 512

NC = 2
NS = 16
NW = NC * NS
GPW = NUM_GRAPHS // NW
LANES = 16
FPR = HIDDEN_DIM // LANES

CH = 128
EFF = CH - 8
NCAP = NUM_NODES - CH

NGROUPS = NUM_NODES // LANES
GQ, GR = divmod(NGROUPS, NW)
SCG = 8
SCROWS = SCG * LANES

NEG_INF = float("-inf")
BIGF = 3.402823466e38


def _bcast(s):
  return lax.broadcast_in_dim(s, (LANES,), ())


def _seg_kernel(x_hbm, batch_hbm, starts_hbm, emb_out, cnt_out,
                xbuf, gmaxbuf, cntbuf, startsbuf):
  del batch_hbm
  wid = lax.axis_index("c") * NS + lax.axis_index("s")
  g_base = wid * GPW

  pltpu.sync_copy(starts_hbm, startsbuf)

  iota = lax.iota(jnp.int32, LANES)
  perms = [jnp.bitwise_xor(iota, 1 << k) for k in range(4)]

  iota_f = jnp.asarray(iota, jnp.float32)

  def per_graph(i, cnt_vec):
    g = g_base + i
    sl = startsbuf[pl.ds(g, LANES)]
    s = sl[0]
    e = sl[1]
    nk = lax.div(e - s + (EFF - 1), EFF)

    def p1_chunk(k, acc):
      pos = s + k * EFF
      base = pl.multiple_of(
          jnp.bitwise_and(jnp.minimum(pos, NCAP), -8), 8)
      pltpu.sync_copy(x_hbm.at[pl.ds(base, CH)], xbuf)
      hi = jnp.minimum(pos + EFF, e)

      def p1_row(j, acc):
        row = base + j
        valid = jnp.logical_and(row >= pos, row < hi)
        mfv = _bcast(jnp.where(valid, 1.0, 0.0))
        off = _bcast(jnp.where(valid, 0.0, NEG_INF))
        out = []
        for c in range(FPR):
          xv = xbuf[j, pl.ds(c * LANES, LANES)]
          out.append(jnp.maximum(acc[c], xv * mfv + off))
        return tuple(out)

      return lax.fori_loop(0, CH, p1_row, acc)

    acc0 = tuple(jnp.full((LANES,), NEG_INF, jnp.float32) for _ in range(FPR))
    acc = lax.fori_loop(0, nk, p1_chunk, acc0)
    for c in range(FPR):
      gmaxbuf[i, pl.ds(c * LANES, LANES)] = acc[c]

    def p2_chunk(k, cnt):
      pos = s + k * EFF
      base = pl.multiple_of(
          jnp.bitwise_and(jnp.minimum(pos, NCAP), -8), 8)
      pltpu.sync_copy(x_hbm.at[pl.ds(base, CH)], xbuf)
      hi = jnp.minimum(pos + EFF, e)

      def p2_row(j, cnt):
        row = base + j
        valid = jnp.logical_and(row >= pos, row < hi)
        d = jnp.full((LANES,), BIGF, jnp.float32)
        for c in range(FPR):
          xv = xbuf[j, pl.ds(c * LANES, LANES)]
          d = jnp.minimum(d, jnp.abs(acc[c] - xv))
        for p in perms:
          d = jnp.minimum(d, d[p])
        matchv = 1.0 - jnp.minimum(d * BIGF * BIGF, 1.0)
        return cnt + matchv * _bcast(jnp.where(valid, 1.0, 0.0))

      return lax.fori_loop(0, CH, p2_row, cnt)

    cnt = lax.fori_loop(0, nk, p2_chunk, jnp.zeros((LANES,), jnp.float32))
    cnt = jnp.maximum(cnt, 1.0)
    onehot = jnp.maximum(1.0 - jnp.abs(iota_f - _bcast(i.astype(jnp.float32))),
                         0.0)
    return cnt_vec + onehot * cnt

  cnt_vec = lax.fori_loop(0, GPW, per_graph, jnp.zeros((LANES,), jnp.float32))
  cntbuf[...] = cnt_vec
  gb = pl.multiple_of(g_base, 8)
  pltpu.sync_copy(gmaxbuf, emb_out.at[pl.ds(gb, GPW)])
  pltpu.sync_copy(cntbuf, cnt_out.at[pl.ds(gb, GPW)])


def _score_kernel(x_hbm, batch_hbm, emb_hbm, cnt_hbm, scores_out,
                  xbuf, bbuf, embbuf, cntbuf, scorebuf):
  wid = lax.axis_index("c") * NS + lax.axis_index("s")
  g0 = wid * GQ + jnp.minimum(wid, GR)
  ng = GQ + jnp.where(wid < GR, 1, 0)
  nsc = lax.div(ng + (SCG - 1), SCG)

  pltpu.sync_copy(emb_hbm, embbuf)
  pltpu.sync_copy(cnt_hbm, cntbuf)

  iota = lax.iota(jnp.int32, LANES)

  def per_chunk(k, carry):
    rbase = pl.multiple_of(
        jnp.minimum(g0 * LANES + k * SCROWS, NUM_NODES - SCROWS), 16)
    pltpu.sync_copy(x_hbm.at[pl.ds(rbase, SCROWS)], xbuf)
    pltpu.sync_copy(batch_hbm.at[pl.ds(rbase, SCROWS)], bbuf)
    for j in range(SCG):
      rowv = iota + j * LANES
      bvec = bbuf[pl.ds(j * LANES, LANES)]

      def col_step(c8, d):
        for u in range(8):
          cvec = _bcast(c8 * 8 + u)
          xg = plsc.load_gather(xbuf, [rowv, cvec])
          eg = plsc.load_gather(embbuf, [bvec, cvec])
          d = jnp.minimum(d, jnp.abs(xg - eg))
        return d

      d = lax.fori_loop(0, HIDDEN_DIM // 8, col_step,
                        jnp.full((LANES,), BIGF, jnp.float32))
      match = 1.0 - jnp.minimum(d * BIGF * BIGF, 1.0)
      cv = plsc.load_gather(cntbuf, [bvec])
      scorebuf[pl.ds(j * LANES, LANES)] = match / cv
    pltpu.sync_copy(scorebuf, scores_out.at[pl.ds(rbase, SCROWS)])
    return carry

  lax.fori_loop(0, nsc, per_chunk, jnp.int32(0))


def _mesh():
  return plsc.VectorSubcoreMesh(
      core_axis_name="c", subcore_axis_name="s", num_cores=NC, num_subcores=NS)


@jax.jit
def kernel(x, batch):
  batch = batch.astype(jnp.int32)
  starts = jnp.searchsorted(batch, jnp.arange(NUM_GRAPHS + 1, dtype=jnp.int32)
                            ).astype(jnp.int32)
  starts = jnp.concatenate(
      [starts, jnp.full((15,), NUM_NODES, jnp.int32)])

  seg = pl.kernel(
      _seg_kernel,
      out_type=(
          jax.ShapeDtypeStruct((NUM_GRAPHS, HIDDEN_DIM), jnp.float32),
          jax.ShapeDtypeStruct((NUM_GRAPHS,), jnp.float32),
      ),
      mesh=_mesh(),
      compiler_params=pltpu.CompilerParams(needs_layout_passes=False),
      scratch_types=[
          pltpu.VMEM((CH, HIDDEN_DIM), jnp.float32),
          pltpu.VMEM((GPW, HIDDEN_DIM), jnp.float32),
          pltpu.VMEM((GPW,), jnp.float32),
          pltpu.VMEM((NUM_GRAPHS + 16,), jnp.int32),
      ],
  )
  emb, cnt = seg(x, batch, starts)

  score = pl.kernel(
      _score_kernel,
      out_type=jax.ShapeDtypeStruct((NUM_NODES,), jnp.float32),
      mesh=_mesh(),
      compiler_params=pltpu.CompilerParams(needs_layout_passes=False),
      scratch_types=[
          pltpu.VMEM((SCROWS, HIDDEN_DIM), jnp.float32),
          pltpu.VMEM((SCROWS,), jnp.int32),
          pltpu.VMEM((NUM_GRAPHS, HIDDEN_DIM), jnp.float32),
          pltpu.VMEM((NUM_GRAPHS,), jnp.float32),
          pltpu.VMEM((SCROWS,), jnp.float32),
      ],
  )
  scores = score(x, batch, emb, cnt)
  return (emb, scores)


# speedup vs baseline: 3.0034x; 1.1629x over previous
import jax
import jax.numpy as jnp
from jax import lax
from jax.experimental import pallas as pl
from jax.experimental.pallas import tpu as pltpu
from jax.experimental.pallas import tpu_sc as plsc

NUM_NODES = 100000
HIDDEN_DIM = 128
NUM_GRAPHS = 512

NC = 2
NS = 16
NW = NC * NS
GPW = NUM_GRAPHS // NW
LANES = 16
FPR = HIDDEN_DIM // LANES

CH = 240
EFF = CH - 8
NCAP = NUM_NODES - CH

NGROUPS = NUM_NODES // LANES
GQ, GR = divmod(NGROUPS, NW)
SCG = 8
SCROWS = SCG * LANES

NEG_INF = float("-inf")
BIGF = 3.402823466e38


def _bcast(s):
  return lax.broadcast_in_dim(s, (LANES,), ())


def _seg_kernel(x_hbm, batch_hbm, starts_hbm, emb_out, cnt_out,
                xbuf, gmaxbuf, cntbuf, startsbuf):
  del batch_hbm
  wid = lax.axis_index("c") * NS + lax.axis_index("s")
  g_base = wid * GPW

  pltpu.sync_copy(starts_hbm, startsbuf)

  iota = lax.iota(jnp.int32, LANES)
  perms = [jnp.bitwise_xor(iota, 1 << k) for k in range(4)]

  iota_f = jnp.asarray(iota, jnp.float32)

  def per_graph(i, cnt_vec):
    g = g_base + i
    sl = startsbuf[pl.ds(g, LANES)]
    s = sl[0]
    e = sl[1]
    nk = lax.div(e - s + (EFF - 1), EFF)

    def p1_chunk(k, acc):
      pos = s + k * EFF
      base = pl.multiple_of(
          jnp.bitwise_and(jnp.minimum(pos, NCAP), -8), 8)
      pltpu.sync_copy(x_hbm.at[pl.ds(base, CH)], xbuf)
      lo = pos - base
      hi = jnp.minimum(pos + EFF, e) - base

      def p1_row(j, acc):
        out = []
        for c in range(FPR):
          xv = xbuf[j, pl.ds(c * LANES, LANES)]
          out.append(jnp.maximum(acc[c], xv))
        return tuple(out)

      return lax.fori_loop(lo, hi, p1_row, acc)

    acc0 = tuple(jnp.full((LANES,), NEG_INF, jnp.float32) for _ in range(FPR))
    acc = lax.fori_loop(0, nk, p1_chunk, acc0)
    for c in range(FPR):
      gmaxbuf[i, pl.ds(c * LANES, LANES)] = acc[c]

    def p2_chunk(k, cnt):
      pos = s + k * EFF
      base = pl.multiple_of(
          jnp.bitwise_and(jnp.minimum(pos, NCAP), -8), 8)
      @pl.when(jnp.logical_or(k > 0, nk > 1))
      def _():
        pltpu.sync_copy(x_hbm.at[pl.ds(base, CH)], xbuf)
      lo = pos - base
      hi = jnp.minimum(pos + EFF, e) - base

      def p2_row(j, cnt):
        d = jnp.full((LANES,), -BIGF, jnp.float32)
        for c in range(FPR):
          xv = xbuf[j, pl.ds(c * LANES, LANES)]
          d = jnp.maximum(d, xv - acc[c])
        for p in perms:
          d = jnp.maximum(d, d[p])
        return cnt + (1.0 - jnp.minimum(-d * BIGF * BIGF, 1.0))

      return lax.fori_loop(lo, hi, p2_row, cnt)

    cnt = lax.fori_loop(0, nk, p2_chunk, jnp.zeros((LANES,), jnp.float32))
    cnt = jnp.maximum(cnt, 1.0)
    onehot = jnp.maximum(1.0 - jnp.abs(iota_f - _bcast(i.astype(jnp.float32))),
                         0.0)
    return cnt_vec + onehot * cnt

  cnt_vec = lax.fori_loop(0, GPW, per_graph, jnp.zeros((LANES,), jnp.float32))
  cntbuf[...] = cnt_vec
  gb = pl.multiple_of(g_base, 8)
  pltpu.sync_copy(gmaxbuf, emb_out.at[pl.ds(gb, GPW)])
  pltpu.sync_copy(cntbuf, cnt_out.at[pl.ds(gb, GPW)])


def _score_kernel(x_hbm, batch_hbm, emb_hbm, cnt_hbm, scores_out,
                  xbuf, bbuf, embbuf, cntbuf, scorebuf):
  wid = lax.axis_index("c") * NS + lax.axis_index("s")
  g0 = wid * GQ + jnp.minimum(wid, GR)
  ng = GQ + jnp.where(wid < GR, 1, 0)
  nsc = lax.div(ng + (SCG - 1), SCG)

  pltpu.sync_copy(emb_hbm, embbuf)
  pltpu.sync_copy(cnt_hbm, cntbuf)

  iota = lax.iota(jnp.int32, LANES)

  def per_chunk(k, carry):
    rbase = pl.multiple_of(
        jnp.minimum(g0 * LANES + k * SCROWS, NUM_NODES - SCROWS), 16)
    pltpu.sync_copy(x_hbm.at[pl.ds(rbase, SCROWS)], xbuf)
    pltpu.sync_copy(batch_hbm.at[pl.ds(rbase, SCROWS)], bbuf)
    for j in range(SCG):
      rowv = iota + j * LANES
      bvec = bbuf[pl.ds(j * LANES, LANES)]

      def col_step(c8, d):
        for u in range(8):
          cvec = _bcast(c8 * 8 + u)
          xg = plsc.load_gather(xbuf, [rowv, cvec])
          eg = plsc.load_gather(embbuf, [bvec, cvec])
          d = jnp.minimum(d, jnp.abs(xg - eg))
        return d

      d = lax.fori_loop(0, HIDDEN_DIM // 8, col_step,
                        jnp.full((LANES,), BIGF, jnp.float32))
      match = 1.0 - jnp.minimum(d * BIGF * BIGF, 1.0)
      cv = plsc.load_gather(cntbuf, [bvec])
      scorebuf[pl.ds(j * LANES, LANES)] = match / cv
    pltpu.sync_copy(scorebuf, scores_out.at[pl.ds(rbase, SCROWS)])
    return carry

  lax.fori_loop(0, nsc, per_chunk, jnp.int32(0))


def _mesh():
  return plsc.VectorSubcoreMesh(
      core_axis_name="c", subcore_axis_name="s", num_cores=NC, num_subcores=NS)


@jax.jit
def kernel(x, batch):
  batch = batch.astype(jnp.int32)
  starts = jnp.searchsorted(batch, jnp.arange(NUM_GRAPHS + 1, dtype=jnp.int32)
                            ).astype(jnp.int32)
  starts = jnp.concatenate(
      [starts, jnp.full((15,), NUM_NODES, jnp.int32)])

  seg = pl.kernel(
      _seg_kernel,
      out_type=(
          jax.ShapeDtypeStruct((NUM_GRAPHS, HIDDEN_DIM), jnp.float32),
          jax.ShapeDtypeStruct((NUM_GRAPHS,), jnp.float32),
      ),
      mesh=_mesh(),
      compiler_params=pltpu.CompilerParams(needs_layout_passes=False),
      scratch_types=[
          pltpu.VMEM((CH, HIDDEN_DIM), jnp.float32),
          pltpu.VMEM((GPW, HIDDEN_DIM), jnp.float32),
          pltpu.VMEM((GPW,), jnp.float32),
          pltpu.VMEM((NUM_GRAPHS + 16,), jnp.int32),
      ],
  )
  emb, cnt = seg(x, batch, starts)

  score = pl.kernel(
      _score_kernel,
      out_type=jax.ShapeDtypeStruct((NUM_NODES,), jnp.float32),
      mesh=_mesh(),
      compiler_params=pltpu.CompilerParams(needs_layout_passes=False),
      scratch_types=[
          pltpu.VMEM((SCROWS, HIDDEN_DIM), jnp.float32),
          pltpu.VMEM((SCROWS,), jnp.int32),
          pltpu.VMEM((NUM_GRAPHS, HIDDEN_DIM), jnp.float32),
          pltpu.VMEM((NUM_GRAPHS,), jnp.float32),
          pltpu.VMEM((SCROWS,), jnp.float32),
      ],
  )
  scores = score(x, batch, emb, cnt)
  return (emb, scores)
